# Initial kernel scaffold; baseline (speedup 1.0000x reference)
import jax
import jax.numpy as jnp
from jax.experimental import pallas as pl


def kernel(x, edge_index, W_gcn, b_gcn, W1, b1, W2, b2, W3, b3):
    raise NotImplementedError("write your pallas kernel here")


# speedup vs baseline: 30.0012x; 30.0012x over previous
import functools

import jax
import jax.numpy as jnp
from jax import lax
from jax.experimental import pallas as pl
from jax.experimental.pallas import tpu as pltpu
from jax.experimental.pallas import tpu_sc as plsc


def _zero_vmem_1d(ref, nwords):
    z = jnp.zeros((16,), jnp.float32)
    def body(i, _):
        ref[pl.ds(i * 16, 16)] = z
        return 0
    lax.fori_loop(0, nwords // 16, body, 0)


def _deg_kernel(n_acc, nch, ch, rpt):
    mesh = plsc.VectorSubcoreMesh(core_axis_name="c", subcore_axis_name="s")
    nch2 = nch // 2

    @functools.partial(
        pl.kernel,
        out_type=jax.ShapeDtypeStruct((2, n_acc), jnp.float32),
        mesh=mesh,
        scratch_types=[
            pltpu.VMEM((nch2, ch), jnp.int32),
            pltpu.VMEM((ch,), jnp.float32),
            pltpu.VMEM((rpt,), jnp.float32),
            pltpu.VMEM_SHARED((n_acc,), jnp.float32),
            pltpu.SemaphoreType.DMA,
        ],
        compiler_params=pltpu.CompilerParams(use_tc_tiling_on_sc=False),
    )
    def deg(dst_hbm, out_hbm, idx_v, ones_v, zer_v, hist, sem):
        c = lax.axis_index("c")
        s = lax.axis_index("s")
        pltpu.sync_copy(dst_hbm.at[s, pl.ds(c * nch2, nch2)], idx_v)
        one = jnp.ones((16,), jnp.float32)
        for i in range(ch // 16):
            ones_v[pl.ds(i * 16, 16)] = one
        _zero_vmem_1d(zer_v, rpt)
        pltpu.sync_copy(zer_v, hist.at[pl.ds(s * rpt, rpt)])
        plsc.subcore_barrier()
        def step(j, _):
            pltpu.sync_copy(ones_v, hist.at[idx_v.at[j]], add=True)
            return 0
        lax.fori_loop(0, nch2, step, 0)
        plsc.subcore_barrier()
        pltpu.sync_copy(
            hist.at[pl.ds(s * rpt, rpt)],
            out_hbm.at[c, pl.ds(s * rpt, rpt)],
        )

    return deg


def _edge_kernel(n_acc, hd, nch, ch, rpt):
    mesh = plsc.VectorSubcoreMesh(core_axis_name="c", subcore_axis_name="s")

    @functools.partial(
        pl.kernel,
        out_type=jax.ShapeDtypeStruct((2, n_acc, hd), jnp.float32),
        mesh=mesh,
        scratch_types=[
            pltpu.VMEM((nch, ch), jnp.int32),
            pltpu.VMEM((nch, ch), jnp.int32),
            pltpu.VMEM((2, ch, hd), jnp.float32),
            pltpu.VMEM_SHARED((n_acc, hd), jnp.float32),
            pltpu.SemaphoreType.DMA,
            pltpu.SemaphoreType.DMA,
        ],
        compiler_params=pltpu.CompilerParams(use_tc_tiling_on_sc=False),
    )
    def edge(y2_hbm, src_hbm, dst_hbm, out_hbm, src_v, dst_v, rows_v, acc,
             sem0, sem1):
        c = lax.axis_index("c")
        s = lax.axis_index("s")
        pltpu.sync_copy(src_hbm.at[s], src_v)
        pltpu.sync_copy(dst_hbm.at[s], dst_v)
        yh = y2_hbm.at[c]
        z = jnp.zeros((16,), jnp.float32)
        def zbody(i, _):
            for k in range(hd // 16):
                rows_v[0, i, pl.ds(k * 16, 16)] = z
            return 0
        lax.fori_loop(0, ch, zbody, 0)
        off = 0
        while off + ch <= rpt:
            pltpu.sync_copy(rows_v.at[0], acc.at[pl.ds(s * rpt + off, ch)])
            off += ch
        rem = rpt - off
        if rem:
            pltpu.sync_copy(
                rows_v.at[0, pl.ds(0, rem)], acc.at[pl.ds(s * rpt + off, rem)])
        plsc.subcore_barrier()

        pltpu.async_copy(yh.at[src_v.at[0]], rows_v.at[0], sem0)
        def step(i, _):
            j0 = i * 2
            pltpu.async_copy(yh.at[src_v.at[j0 + 1]], rows_v.at[1], sem1)
            pltpu.make_async_copy(
                yh.at[src_v.at[j0]], rows_v.at[0], sem0).wait()
            pltpu.sync_copy(rows_v.at[0], acc.at[dst_v.at[j0]], add=True)
            pltpu.async_copy(yh.at[src_v.at[j0 + 2]], rows_v.at[0], sem0)
            pltpu.make_async_copy(
                yh.at[src_v.at[j0 + 1]], rows_v.at[1], sem1).wait()
            pltpu.sync_copy(rows_v.at[1], acc.at[dst_v.at[j0 + 1]], add=True)
            return 0
        lax.fori_loop(0, (nch - 2) // 2, step, 0)
        pltpu.async_copy(yh.at[src_v.at[nch - 1]], rows_v.at[1], sem1)
        pltpu.make_async_copy(
            yh.at[src_v.at[nch - 2]], rows_v.at[0], sem0).wait()
        pltpu.sync_copy(rows_v.at[0], acc.at[dst_v.at[nch - 2]], add=True)
        pltpu.make_async_copy(
            yh.at[src_v.at[nch - 1]], rows_v.at[1], sem1).wait()
        pltpu.sync_copy(rows_v.at[1], acc.at[dst_v.at[nch - 1]], add=True)

        plsc.subcore_barrier()
        pltpu.sync_copy(
            acc.at[pl.ds(s * rpt, rpt)],
            out_hbm.at[c, pl.ds(s * rpt, rpt)],
        )

    return edge


def _tc_scale_body(x_ref, w_ref, dp_ref, y_ref):
    deg = dp_ref[0] + dp_ref[1] + 1.0
    dinv = lax.rsqrt(deg)
    xw = jnp.dot(x_ref[...], w_ref[...], preferred_element_type=jnp.float32)
    y = xw * dinv
    hd = y.shape[-1] // 2
    y_ref[0] = y[:, :hd]
    y_ref[1] = y[:, hd:]


def _tc_head_body(acc_ref, y_ref, x_ref, dp_ref, bg_ref, w1_ref, b1_ref,
                  w2_ref, b2_ref, w3t_ref, b3_ref, o_ref):
    deg = dp_ref[0] + dp_ref[1] + 1.0
    dinv = lax.rsqrt(deg)
    acc = jnp.concatenate([acc_ref[0], acc_ref[1]], axis=1)
    y = jnp.concatenate([y_ref[0], y_ref[1]], axis=1)
    tot = (acc + y) * dinv + bg_ref[...]
    h = jnp.maximum(tot, 0.0) + x_ref[...]
    h1 = jnp.dot(h, w1_ref[...], preferred_element_type=jnp.float32) + b1_ref[...]
    h1 = jnp.where(h1 >= 0.0, h1, 0.01 * h1)
    h2 = jnp.dot(h1, w2_ref[...], preferred_element_type=jnp.float32) + b2_ref[...]
    h2 = jnp.where(h2 >= 0.0, h2, 0.01 * h2)
    o_ref[...] = jnp.sum(h2 * w3t_ref[...], axis=1, keepdims=True) + b3_ref[...]


def kernel(x, edge_index, W_gcn, b_gcn, W1, b1, W2, b2, W3, b3):
    n, d = x.shape
    e = edge_index.shape[1]
    hd = d // 2
    ng = 16
    ch = 128
    ep = e // ng
    nch = 16 * (-(-ep // (16 * ch)))
    epp = nch * ch
    rpt = 640
    n_acc = 16 * rpt
    act = 8

    src = edge_index[0].astype(jnp.int32).reshape(ng, ep)
    dst = edge_index[1].astype(jnp.int32).reshape(ng, ep)
    npad = epp - ep
    if npad:
        pad_src = (jnp.arange(npad, dtype=jnp.int32) * 97) % n
        pad_dst = n + (jnp.arange(npad, dtype=jnp.int32) % (n_acc - n))
        src = jnp.concatenate(
            [src, jnp.broadcast_to(pad_src, (ng, npad))], axis=1)
        dst = jnp.concatenate(
            [dst, jnp.broadcast_to(pad_dst, (ng, npad))], axis=1)
    src = src.reshape(ng, nch, ch)
    dst = dst.reshape(ng, nch, ch)

    deg_p = _deg_kernel(n_acc, nch, ch, rpt)(dst)
    deg3 = deg_p.reshape(2, n_acc, 1)

    blk = 1000
    grid = n // blk
    y2 = pl.pallas_call(
        _tc_scale_body,
        grid=(grid,),
        in_specs=[
            pl.BlockSpec((blk, d), lambda i: (i, 0)),
            pl.BlockSpec((d, d), lambda i: (0, 0)),
            pl.BlockSpec((2, blk, 1), lambda i: (0, i, 0)),
        ],
        out_specs=pl.BlockSpec((2, blk, hd), lambda i: (0, i, 0)),
        out_shape=jax.ShapeDtypeStruct((2, n, hd), jnp.float32),
    )(x, W_gcn, deg3)

    acc = _edge_kernel(n_acc, hd, nch, ch, rpt)(y2, src, dst)

    out = pl.pallas_call(
        _tc_head_body,
        grid=(grid,),
        in_specs=[
            pl.BlockSpec((2, blk, hd), lambda i: (0, i, 0)),
            pl.BlockSpec((2, blk, hd), lambda i: (0, i, 0)),
            pl.BlockSpec((blk, d), lambda i: (i, 0)),
            pl.BlockSpec((2, blk, 1), lambda i: (0, i, 0)),
            pl.BlockSpec((1, d), lambda i: (0, 0)),
            pl.BlockSpec((d, 32), lambda i: (0, 0)),
            pl.BlockSpec((1, 32), lambda i: (0, 0)),
            pl.BlockSpec((32, 32), lambda i: (0, 0)),
            pl.BlockSpec((1, 32), lambda i: (0, 0)),
            pl.BlockSpec((1, 32), lambda i: (0, 0)),
            pl.BlockSpec((1, 1), lambda i: (0, 0)),
        ],
        out_specs=pl.BlockSpec((blk, 1), lambda i: (i, 0)),
        out_shape=jax.ShapeDtypeStruct((n, 1), jnp.float32),
    )(acc, y2, x, deg3, b_gcn.reshape(1, d), W1, b1.reshape(1, 32),
      W2, b2.reshape(1, 32), W3.reshape(1, 32), b3.reshape(1, 1))

    return out.reshape(n // act, act, 1)


# speedup vs baseline: 31.4277x; 1.0475x over previous
import functools

import jax
import jax.numpy as jnp
from jax import lax
from jax.experimental import pallas as pl
from jax.experimental.pallas import tpu as pltpu
from jax.experimental.pallas import tpu_sc as plsc


def _zero_vmem_1d(ref, nwords):
    z = jnp.zeros((16,), jnp.float32)
    def body(i, _):
        ref[pl.ds(i * 16, 16)] = z
        return 0
    lax.fori_loop(0, nwords // 16, body, 0)


def _deg_kernel(n_acc, nch, ch, rpt):
    mesh = plsc.VectorSubcoreMesh(core_axis_name="c", subcore_axis_name="s")
    nch2 = nch // 2

    @functools.partial(
        pl.kernel,
        out_type=jax.ShapeDtypeStruct((2, n_acc), jnp.float32),
        mesh=mesh,
        scratch_types=[
            pltpu.VMEM((nch2, ch), jnp.int32),
            pltpu.VMEM((ch,), jnp.float32),
            pltpu.VMEM((rpt,), jnp.float32),
            pltpu.VMEM_SHARED((n_acc,), jnp.float32),
            pltpu.SemaphoreType.DMA,
        ],
        compiler_params=pltpu.CompilerParams(use_tc_tiling_on_sc=False),
    )
    def deg(dst_hbm, out_hbm, idx_v, ones_v, zer_v, hist, sem):
        c = lax.axis_index("c")
        s = lax.axis_index("s")
        pltpu.sync_copy(dst_hbm.at[s, pl.ds(c * nch2, nch2)], idx_v)
        one = jnp.ones((16,), jnp.float32)
        for i in range(ch // 16):
            ones_v[pl.ds(i * 16, 16)] = one
        _zero_vmem_1d(zer_v, rpt)
        pltpu.sync_copy(zer_v, hist.at[pl.ds(s * rpt, rpt)])
        plsc.subcore_barrier()
        def step(j, _):
            pltpu.sync_copy(ones_v, hist.at[idx_v.at[j]], add=True)
            return 0
        lax.fori_loop(0, nch2, step, 0)
        plsc.subcore_barrier()
        pltpu.sync_copy(
            hist.at[pl.ds(s * rpt, rpt)],
            out_hbm.at[c, pl.ds(s * rpt, rpt)],
        )

    return deg


def _edge_kernel(n_acc, hd, nch, ch, rpt):
    mesh = plsc.VectorSubcoreMesh(core_axis_name="c", subcore_axis_name="s")
    nchh = nch // 2

    @functools.partial(
        pl.kernel,
        out_type=jax.ShapeDtypeStruct((2, n_acc, hd), jnp.float32),
        mesh=mesh,
        scratch_types=[
            pltpu.VMEM((nchh, ch), jnp.int32),
            pltpu.VMEM((nchh, ch), jnp.int32),
            pltpu.VMEM((8, ch, hd), jnp.float32),
            pltpu.VMEM_SHARED((n_acc, hd), jnp.float32),
            pltpu.SemaphoreType.DMA,
            pltpu.SemaphoreType.DMA,
            pltpu.SemaphoreType.DMA,
            pltpu.SemaphoreType.DMA,
        ],
        compiler_params=pltpu.CompilerParams(use_tc_tiling_on_sc=False),
    )
    def edge(y2_hbm, src_hbm, dst_hbm, out_hbm, src_v, dst_v, rows_v, acc,
             sem_ga, sem_gb, sem_sa, sem_sb):
        c = lax.axis_index("c")
        s = lax.axis_index("s")
        yh = y2_hbm.at[c]
        z = jnp.zeros((16,), jnp.float32)
        def zbody(i, _):
            for k in range(hd // 16):
                rows_v[0, i, pl.ds(k * 16, 16)] = z
            return 0
        lax.fori_loop(0, ch, zbody, 0)
        off = 0
        while off + ch <= rpt:
            pltpu.sync_copy(rows_v.at[0], acc.at[pl.ds(s * rpt + off, ch)])
            off += ch
        rem = rpt - off
        if rem:
            pltpu.sync_copy(
                rows_v.at[0, pl.ds(0, rem)], acc.at[pl.ds(s * rpt + off, rem)])
        plsc.subcore_barrier()

        def fire_g(j0, grp, sem):
            for b in range(4):
                pltpu.async_copy(
                    yh.at[src_v.at[j0 + b]], rows_v.at[grp * 4 + b], sem)

        def drain_g(j0, grp, sem):
            for b in range(4):
                pltpu.make_async_copy(
                    yh.at[src_v.at[j0 + b]], rows_v.at[grp * 4 + b], sem).wait()

        def fire_s(j0, grp, sem):
            for b in range(4):
                pltpu.async_copy(
                    rows_v.at[grp * 4 + b], acc.at[dst_v.at[j0 + b]], sem,
                    add=True)

        def drain_s(j0, grp, sem):
            for b in range(4):
                pltpu.make_async_copy(
                    rows_v.at[grp * 4 + b], acc.at[dst_v.at[j0 + b]], sem
                ).wait()

        for h in range(2):
            pltpu.sync_copy(src_hbm.at[s, pl.ds(h * nchh, nchh)], src_v)
            pltpu.sync_copy(dst_hbm.at[s, pl.ds(h * nchh, nchh)], dst_v)
            fire_g(0, 0, sem_ga)
            def step(m, _):
                ja = m * 8
                jb = ja + 4
                drain_g(ja, 0, sem_ga)
                fire_s(ja, 0, sem_sa)
                @pl.when(m > 0)
                def _():
                    drain_s(ja - 4, 1, sem_sb)
                fire_g(jb, 1, sem_gb)
                drain_g(jb, 1, sem_gb)
                fire_s(jb, 1, sem_sb)
                drain_s(ja, 0, sem_sa)
                fire_g(jb + 4, 0, sem_ga)
                return 0
            lax.fori_loop(0, nchh // 8 - 1, step, 0)
            ja = nchh - 8
            jb = nchh - 4
            drain_g(ja, 0, sem_ga)
            fire_s(ja, 0, sem_sa)
            drain_s(ja - 4, 1, sem_sb)
            fire_g(jb, 1, sem_gb)
            drain_g(jb, 1, sem_gb)
            fire_s(jb, 1, sem_sb)
            drain_s(ja, 0, sem_sa)
            drain_s(jb, 1, sem_sb)

        plsc.subcore_barrier()
        pltpu.sync_copy(
            acc.at[pl.ds(s * rpt, rpt)],
            out_hbm.at[c, pl.ds(s * rpt, rpt)],
        )

    return edge


def _tc_scale_body(x_ref, w_ref, dp_ref, y_ref):
    deg = dp_ref[0] + dp_ref[1] + 1.0
    dinv = lax.rsqrt(deg)
    xw = jnp.dot(x_ref[...], w_ref[...], preferred_element_type=jnp.float32)
    y = xw * dinv
    hd = y.shape[-1] // 2
    y_ref[0] = y[:, :hd]
    y_ref[1] = y[:, hd:]


def _tc_head_body(acc_ref, y_ref, x_ref, dp_ref, bg_ref, w1_ref, b1_ref,
                  w2_ref, b2_ref, w3t_ref, b3_ref, o_ref):
    deg = dp_ref[0] + dp_ref[1] + 1.0
    dinv = lax.rsqrt(deg)
    acc = jnp.concatenate([acc_ref[0], acc_ref[1]], axis=1)
    y = jnp.concatenate([y_ref[0], y_ref[1]], axis=1)
    tot = (acc + y) * dinv + bg_ref[...]
    h = jnp.maximum(tot, 0.0) + x_ref[...]
    h1 = jnp.dot(h, w1_ref[...], preferred_element_type=jnp.float32) + b1_ref[...]
    h1 = jnp.where(h1 >= 0.0, h1, 0.01 * h1)
    h2 = jnp.dot(h1, w2_ref[...], preferred_element_type=jnp.float32) + b2_ref[...]
    h2 = jnp.where(h2 >= 0.0, h2, 0.01 * h2)
    o_ref[...] = jnp.sum(h2 * w3t_ref[...], axis=1, keepdims=True) + b3_ref[...]


def kernel(x, edge_index, W_gcn, b_gcn, W1, b1, W2, b2, W3, b3):
    n, d = x.shape
    e = edge_index.shape[1]
    hd = d // 2
    ng = 16
    ch = 128
    ep = e // ng
    nch = 16 * (-(-ep // (16 * ch)))
    epp = nch * ch
    rpt = 640
    n_acc = 16 * rpt
    act = 8

    src = edge_index[0].astype(jnp.int32).reshape(ng, ep)
    dst = edge_index[1].astype(jnp.int32).reshape(ng, ep)
    npad = epp - ep
    if npad:
        pad_src = (jnp.arange(npad, dtype=jnp.int32) * 97) % n
        pad_dst = n + (jnp.arange(npad, dtype=jnp.int32) % (n_acc - n))
        src = jnp.concatenate(
            [src, jnp.broadcast_to(pad_src, (ng, npad))], axis=1)
        dst = jnp.concatenate(
            [dst, jnp.broadcast_to(pad_dst, (ng, npad))], axis=1)
    src = src.reshape(ng, nch, ch)
    dst = dst.reshape(ng, nch, ch)

    deg_p = _deg_kernel(n_acc, nch, ch, rpt)(dst)
    deg3 = deg_p.reshape(2, n_acc, 1)

    blk = 1000
    grid = n // blk
    y2 = pl.pallas_call(
        _tc_scale_body,
        grid=(grid,),
        in_specs=[
            pl.BlockSpec((blk, d), lambda i: (i, 0)),
            pl.BlockSpec((d, d), lambda i: (0, 0)),
            pl.BlockSpec((2, blk, 1), lambda i: (0, i, 0)),
        ],
        out_specs=pl.BlockSpec((2, blk, hd), lambda i: (0, i, 0)),
        out_shape=jax.ShapeDtypeStruct((2, n, hd), jnp.float32),
    )(x, W_gcn, deg3)

    acc = _edge_kernel(n_acc, hd, nch, ch, rpt)(y2, src, dst)

    out = pl.pallas_call(
        _tc_head_body,
        grid=(grid,),
        in_specs=[
            pl.BlockSpec((2, blk, hd), lambda i: (0, i, 0)),
            pl.BlockSpec((2, blk, hd), lambda i: (0, i, 0)),
            pl.BlockSpec((blk, d), lambda i: (i, 0)),
            pl.BlockSpec((2, blk, 1), lambda i: (0, i, 0)),
            pl.BlockSpec((1, d), lambda i: (0, 0)),
            pl.BlockSpec((d, 32), lambda i: (0, 0)),
            pl.BlockSpec((1, 32), lambda i: (0, 0)),
            pl.BlockSpec((32, 32), lambda i: (0, 0)),
            pl.BlockSpec((1, 32), lambda i: (0, 0)),
            pl.BlockSpec((1, 32), lambda i: (0, 0)),
            pl.BlockSpec((1, 1), lambda i: (0, 0)),
        ],
        out_specs=pl.BlockSpec((blk, 1), lambda i: (i, 0)),
        out_shape=jax.ShapeDtypeStruct((n, 1), jnp.float32),
    )(acc, y2, x, deg3, b_gcn.reshape(1, d), W1, b1.reshape(1, 32),
      W2, b2.reshape(1, 32), W3.reshape(1, 32), b3.reshape(1, 1))

    return out.reshape(n // act, act, 1)


# speedup vs baseline: 35.9797x; 1.1448x over previous
import functools

import jax
import jax.numpy as jnp
from jax import lax
from jax.experimental import pallas as pl
from jax.experimental.pallas import tpu as pltpu
from jax.experimental.pallas import tpu_sc as plsc

_CH = 128


def _zero_vmem_1d(ref, nwords):
    z = jnp.zeros((16,), jnp.float32)
    def body(i, _):
        ref[pl.ds(i * 16, 16)] = z
        return 0
    lax.fori_loop(0, nwords // 16, body, 0)


def _deg_kernel(n_acc, nck, rpt):
    mesh = plsc.VectorSubcoreMesh(core_axis_name="c", subcore_axis_name="s")
    per = nck // 16
    half = per // 2
    nextra = nck - per * 16

    @functools.partial(
        pl.kernel,
        out_type=jax.ShapeDtypeStruct((2, n_acc), jnp.float32),
        mesh=mesh,
        scratch_types=[
            pltpu.VMEM((half, _CH), jnp.int32),
            pltpu.VMEM((1, _CH), jnp.int32),
            pltpu.VMEM((_CH,), jnp.float32),
            pltpu.VMEM((rpt,), jnp.float32),
            pltpu.VMEM_SHARED((n_acc,), jnp.float32),
            pltpu.SemaphoreType.DMA,
        ],
        compiler_params=pltpu.CompilerParams(use_tc_tiling_on_sc=False),
    )
    def deg(ei_hbm, out_hbm, idx_v, idx1_v, ones_v, zer_v, hist, sem):
        c = lax.axis_index("c")
        s = lax.axis_index("s")
        base = s * per + c * half
        pltpu.sync_copy(ei_hbm.at[1, pl.ds(base, half)], idx_v)
        one = jnp.ones((16,), jnp.float32)
        for i in range(_CH // 16):
            ones_v[pl.ds(i * 16, 16)] = one
        _zero_vmem_1d(zer_v, rpt)
        pltpu.sync_copy(zer_v, hist.at[pl.ds(s * rpt, rpt)])
        plsc.subcore_barrier()
        def step(j, _):
            pltpu.sync_copy(ones_v, hist.at[idx_v.at[j]], add=True)
            return 0
        lax.fori_loop(0, half, step, 0)
        @pl.when(s < nextra // 2)
        def _():
            pltpu.sync_copy(
                ei_hbm.at[1, pl.ds(16 * per + c * (nextra // 2) + s, 1)],
                idx1_v)
            pltpu.sync_copy(ones_v, hist.at[idx1_v.at[0]], add=True)
        plsc.subcore_barrier()
        pltpu.sync_copy(
            hist.at[pl.ds(s * rpt, rpt)],
            out_hbm.at[c, pl.ds(s * rpt, rpt)],
        )

    return deg


def _edge_kernel(n_acc, hd, nck, rpt):
    mesh = plsc.VectorSubcoreMesh(core_axis_name="c", subcore_axis_name="s")
    per = nck // 16
    half = per // 2
    nextra = nck - per * 16
    K = 3

    @functools.partial(
        pl.kernel,
        out_type=jax.ShapeDtypeStruct((2, n_acc, hd), jnp.float32),
        mesh=mesh,
        scratch_types=[
            pltpu.VMEM((half, _CH), jnp.int32),
            pltpu.VMEM((half, _CH), jnp.int32),
            pltpu.VMEM((1, _CH), jnp.int32),
            pltpu.VMEM((1, _CH), jnp.int32),
            pltpu.VMEM((2 * K, _CH, hd), jnp.float32),
            pltpu.VMEM_SHARED((n_acc, hd), jnp.float32),
            pltpu.SemaphoreType.DMA,
            pltpu.SemaphoreType.DMA,
            pltpu.SemaphoreType.DMA,
            pltpu.SemaphoreType.DMA,
        ],
        compiler_params=pltpu.CompilerParams(use_tc_tiling_on_sc=False),
    )
    def edge(y2_hbm, ei_hbm, out_hbm, src_v, dst_v, srcx_v, dstx_v, rows_v,
             acc, sem_ga, sem_gb, sem_sa, sem_sb):
        c = lax.axis_index("c")
        s = lax.axis_index("s")
        yh = y2_hbm.at[c]
        base = s * per

        def fire_g(j0, grp, sem):
            for b in range(K):
                pltpu.async_copy(
                    yh.at[src_v.at[j0 + b]], rows_v.at[grp * K + b], sem)

        def drain_g(j0, grp, sem):
            for b in range(K):
                pltpu.make_async_copy(
                    yh.at[src_v.at[j0 + b]], rows_v.at[grp * K + b], sem
                ).wait()

        def fire_s(j0, grp, sem):
            for b in range(K):
                pltpu.async_copy(
                    rows_v.at[grp * K + b], acc.at[dst_v.at[j0 + b]], sem,
                    add=True)

        def drain_s(j0, grp, sem):
            for b in range(K):
                pltpu.make_async_copy(
                    rows_v.at[grp * K + b], acc.at[dst_v.at[j0 + b]], sem
                ).wait()

        pltpu.sync_copy(ei_hbm.at[0, pl.ds(base, half)], src_v)
        pltpu.sync_copy(ei_hbm.at[1, pl.ds(base, half)], dst_v)
        fire_g(0, 0, sem_ga)
        pltpu.sync_copy(
            y2_hbm.at[c, pl.ds(s * rpt, rpt)], acc.at[pl.ds(s * rpt, rpt)])
        plsc.subcore_barrier()

        for h in range(2):
            if h:
                pltpu.sync_copy(ei_hbm.at[0, pl.ds(base + half, half)], src_v)
                pltpu.sync_copy(ei_hbm.at[1, pl.ds(base + half, half)], dst_v)
                fire_g(0, 0, sem_ga)
            def step(m, _):
                ja = m * 2 * K
                jb = ja + K
                drain_g(ja, 0, sem_ga)
                fire_s(ja, 0, sem_sa)
                @pl.when(m > 0)
                def _():
                    drain_s(ja - K, 1, sem_sb)
                fire_g(jb, 1, sem_gb)
                drain_g(jb, 1, sem_gb)
                fire_s(jb, 1, sem_sb)
                drain_s(ja, 0, sem_sa)
                fire_g(jb + K, 0, sem_ga)
                return 0
            lax.fori_loop(0, half // (2 * K) - 1, step, 0)
            ja = half - 2 * K
            jb = half - K
            drain_g(ja, 0, sem_ga)
            fire_s(ja, 0, sem_sa)
            drain_s(ja - K, 1, sem_sb)
            fire_g(jb, 1, sem_gb)
            drain_g(jb, 1, sem_gb)
            fire_s(jb, 1, sem_sb)
            drain_s(ja, 0, sem_sa)
            drain_s(jb, 1, sem_sb)

        @pl.when(s < nextra)
        def _():
            pltpu.sync_copy(ei_hbm.at[0, pl.ds(16 * per + s, 1)], srcx_v)
            pltpu.sync_copy(ei_hbm.at[1, pl.ds(16 * per + s, 1)], dstx_v)
            pltpu.async_copy(
                yh.at[srcx_v.at[0]], rows_v.at[0], sem_ga).wait()
            pltpu.sync_copy(rows_v.at[0], acc.at[dstx_v.at[0]], add=True)

        plsc.subcore_barrier()
        pltpu.sync_copy(
            acc.at[pl.ds(s * rpt, rpt)],
            out_hbm.at[c, pl.ds(s * rpt, rpt)],
        )

    return edge


def _tc_mm_body(x_ref, w_ref, xw_ref):
    xw_ref[...] = jnp.dot(
        x_ref[...], w_ref[...], preferred_element_type=jnp.float32)


def _dinv_col(dp_ref, blk):
    dvec = dp_ref[0] + dp_ref[1] + 1.0
    return jnp.transpose(lax.rsqrt(dvec)[None, :])


def _tc_scale_body(xw_ref, dp_ref, y_ref):
    dinv = _dinv_col(dp_ref, xw_ref.shape[0])
    y = xw_ref[...] * dinv
    hd = y.shape[-1] // 2
    y_ref[0] = y[:, :hd]
    y_ref[1] = y[:, hd:]


def _tc_head_body(acc_ref, x_ref, dp_ref, bg_ref, w1_ref, b1_ref,
                  w2_ref, b2_ref, w3t_ref, b3_ref, o_ref):
    dinv = _dinv_col(dp_ref, x_ref.shape[0])
    acc = jnp.concatenate([acc_ref[0], acc_ref[1]], axis=1)
    tot = acc * dinv + bg_ref[...]
    h = jnp.maximum(tot, 0.0) + x_ref[...]
    h1 = jnp.dot(h, w1_ref[...], preferred_element_type=jnp.float32) + b1_ref[...]
    h1 = jnp.where(h1 >= 0.0, h1, 0.01 * h1)
    h2 = jnp.dot(h1, w2_ref[...], preferred_element_type=jnp.float32) + b2_ref[...]
    h2 = jnp.where(h2 >= 0.0, h2, 0.01 * h2)
    o_ref[...] = jnp.sum(h2 * w3t_ref[...], axis=1, keepdims=True) + b3_ref[...]


def kernel(x, edge_index, W_gcn, b_gcn, W1, b1, W2, b2, W3, b3):
    n, d = x.shape
    e = edge_index.shape[1]
    hd = d // 2
    nck = e // _CH
    rpt = 640
    n_acc = 16 * rpt
    act = 8

    ei3 = edge_index.astype(jnp.int32).reshape(2, nck, _CH)

    blk = 2048
    grid = n_acc // blk

    deg_p = _deg_kernel(n_acc, nck, rpt)(ei3)
    xw = pl.pallas_call(
        _tc_mm_body,
        grid=(grid,),
        in_specs=[
            pl.BlockSpec((blk, d), lambda i: (i, 0)),
            pl.BlockSpec((d, d), lambda i: (0, 0)),
        ],
        out_specs=pl.BlockSpec((blk, d), lambda i: (i, 0)),
        out_shape=jax.ShapeDtypeStruct((n, d), jnp.float32),
    )(x, W_gcn)

    y2 = pl.pallas_call(
        _tc_scale_body,
        grid=(grid,),
        in_specs=[
            pl.BlockSpec((blk, d), lambda i: (i, 0)),
            pl.BlockSpec((2, blk), lambda i: (0, i)),
        ],
        out_specs=pl.BlockSpec((2, blk, hd), lambda i: (0, i, 0)),
        out_shape=jax.ShapeDtypeStruct((2, n_acc, hd), jnp.float32),
    )(xw, deg_p)

    acc = _edge_kernel(n_acc, hd, nck, rpt)(y2, ei3)

    out = pl.pallas_call(
        _tc_head_body,
        grid=(grid,),
        in_specs=[
            pl.BlockSpec((2, blk, hd), lambda i: (0, i, 0)),
            pl.BlockSpec((blk, d), lambda i: (i, 0)),
            pl.BlockSpec((2, blk), lambda i: (0, i)),
            pl.BlockSpec((1, d), lambda i: (0, 0)),
            pl.BlockSpec((d, 32), lambda i: (0, 0)),
            pl.BlockSpec((1, 32), lambda i: (0, 0)),
            pl.BlockSpec((32, 32), lambda i: (0, 0)),
            pl.BlockSpec((1, 32), lambda i: (0, 0)),
            pl.BlockSpec((1, 32), lambda i: (0, 0)),
            pl.BlockSpec((1, 1), lambda i: (0, 0)),
        ],
        out_specs=pl.BlockSpec((blk, 1), lambda i: (i, 0)),
        out_shape=jax.ShapeDtypeStruct((n, 1), jnp.float32),
    )(acc, x, deg_p, b_gcn.reshape(1, d), W1, b1.reshape(1, 32),
      W2, b2.reshape(1, 32), W3.reshape(1, 32), b3.reshape(1, 1))

    return out.reshape(n // act, act, 1)


# speedup vs baseline: 37.6953x; 1.0477x over previous
import functools

import jax
import jax.numpy as jnp
from jax import lax
from jax.experimental import pallas as pl
from jax.experimental.pallas import tpu as pltpu
from jax.experimental.pallas import tpu_sc as plsc

_CH = 128


def _zero_vmem_1d(ref, nwords):
    z = jnp.zeros((16,), jnp.float32)
    def body(i, _):
        ref[pl.ds(i * 16, 16)] = z
        return 0
    lax.fori_loop(0, nwords // 16, body, 0)


def _deg_kernel(n_acc, nck, rpt):
    mesh = plsc.VectorSubcoreMesh(core_axis_name="c", subcore_axis_name="s")
    per = nck // 16
    half = per // 2
    nextra = nck - per * 16

    @functools.partial(
        pl.kernel,
        out_type=jax.ShapeDtypeStruct((2, n_acc), jnp.float32),
        mesh=mesh,
        scratch_types=[
            pltpu.VMEM((half, _CH), jnp.int32),
            pltpu.VMEM((1, _CH), jnp.int32),
            pltpu.VMEM((_CH,), jnp.float32),
            pltpu.VMEM((rpt,), jnp.float32),
            pltpu.VMEM_SHARED((n_acc,), jnp.float32),
            pltpu.SemaphoreType.DMA,
        ],
        compiler_params=pltpu.CompilerParams(use_tc_tiling_on_sc=False),
    )
    def deg(ei_hbm, out_hbm, idx_v, idx1_v, ones_v, zer_v, hist, sem):
        c = lax.axis_index("c")
        s = lax.axis_index("s")
        base = s * per + c * half
        pltpu.sync_copy(ei_hbm.at[1, pl.ds(base, half)], idx_v)
        one = jnp.ones((16,), jnp.float32)
        for i in range(_CH // 16):
            ones_v[pl.ds(i * 16, 16)] = one
        _zero_vmem_1d(zer_v, rpt)
        pltpu.sync_copy(zer_v, hist.at[pl.ds(s * rpt, rpt)])
        plsc.subcore_barrier()
        kb = 6
        def step(m, _):
            for b in range(kb):
                pltpu.async_copy(
                    ones_v, hist.at[idx_v.at[m * kb + b]], sem, add=True)
            @pl.when(m > 0)
            def _():
                for b in range(kb):
                    pltpu.make_async_copy(
                        ones_v, hist.at[idx_v.at[b]], sem).wait()
            return 0
        lax.fori_loop(0, half // kb, step, 0)
        for b in range(kb):
            pltpu.make_async_copy(ones_v, hist.at[idx_v.at[b]], sem).wait()
        @pl.when(s < nextra // 2)
        def _():
            pltpu.sync_copy(
                ei_hbm.at[1, pl.ds(16 * per + c * (nextra // 2) + s, 1)],
                idx1_v)
            pltpu.sync_copy(ones_v, hist.at[idx1_v.at[0]], add=True)
        plsc.subcore_barrier()
        pltpu.sync_copy(
            hist.at[pl.ds(s * rpt, rpt)],
            out_hbm.at[c, pl.ds(s * rpt, rpt)],
        )

    return deg


def _edge_kernel(n_acc, hd, nck, rpt):
    mesh = plsc.VectorSubcoreMesh(core_axis_name="c", subcore_axis_name="s")
    per = nck // 16
    half = per // 2
    nextra = nck - per * 16
    K = 3

    @functools.partial(
        pl.kernel,
        out_type=jax.ShapeDtypeStruct((2, n_acc, hd), jnp.float32),
        mesh=mesh,
        scratch_types=[
            pltpu.VMEM((half, _CH), jnp.int32),
            pltpu.VMEM((half, _CH), jnp.int32),
            pltpu.VMEM((1, _CH), jnp.int32),
            pltpu.VMEM((1, _CH), jnp.int32),
            pltpu.VMEM((2 * K, _CH, hd), jnp.float32),
            pltpu.VMEM_SHARED((n_acc, hd), jnp.float32),
            pltpu.SemaphoreType.DMA,
            pltpu.SemaphoreType.DMA,
            pltpu.SemaphoreType.DMA,
            pltpu.SemaphoreType.DMA,
        ],
        compiler_params=pltpu.CompilerParams(use_tc_tiling_on_sc=False),
    )
    def edge(y2_hbm, ei_hbm, out_hbm, src_v, dst_v, srcx_v, dstx_v, rows_v,
             acc, sem_ga, sem_gb, sem_sa, sem_sb):
        c = lax.axis_index("c")
        s = lax.axis_index("s")
        yh = y2_hbm.at[c]
        base = s * per

        def fire_g(j0, grp, sem):
            for b in range(K):
                pltpu.async_copy(
                    yh.at[src_v.at[j0 + b]], rows_v.at[grp * K + b], sem)

        def drain_g(j0, grp, sem):
            for b in range(K):
                pltpu.make_async_copy(
                    yh.at[src_v.at[j0 + b]], rows_v.at[grp * K + b], sem
                ).wait()

        def fire_s(j0, grp, sem):
            for b in range(K):
                pltpu.async_copy(
                    rows_v.at[grp * K + b], acc.at[dst_v.at[j0 + b]], sem,
                    add=True)

        def drain_s(j0, grp, sem):
            for b in range(K):
                pltpu.make_async_copy(
                    rows_v.at[grp * K + b], acc.at[dst_v.at[j0 + b]], sem
                ).wait()

        pltpu.sync_copy(ei_hbm.at[0, pl.ds(base, half)], src_v)
        pltpu.sync_copy(ei_hbm.at[1, pl.ds(base, half)], dst_v)
        fire_g(0, 0, sem_ga)
        pltpu.sync_copy(
            y2_hbm.at[c, pl.ds(s * rpt, rpt)], acc.at[pl.ds(s * rpt, rpt)])
        plsc.subcore_barrier()

        for h in range(2):
            if h:
                pltpu.sync_copy(ei_hbm.at[0, pl.ds(base + half, half)], src_v)
                pltpu.sync_copy(ei_hbm.at[1, pl.ds(base + half, half)], dst_v)
                fire_g(0, 0, sem_ga)
            def step(m, _):
                ja = m * 2 * K
                jb = ja + K
                drain_g(ja, 0, sem_ga)
                fire_s(ja, 0, sem_sa)
                @pl.when(m > 0)
                def _():
                    drain_s(ja - K, 1, sem_sb)
                fire_g(jb, 1, sem_gb)
                drain_g(jb, 1, sem_gb)
                fire_s(jb, 1, sem_sb)
                drain_s(ja, 0, sem_sa)
                fire_g(jb + K, 0, sem_ga)
                return 0
            lax.fori_loop(0, half // (2 * K) - 1, step, 0)
            ja = half - 2 * K
            jb = half - K
            drain_g(ja, 0, sem_ga)
            fire_s(ja, 0, sem_sa)
            drain_s(ja - K, 1, sem_sb)
            fire_g(jb, 1, sem_gb)
            drain_g(jb, 1, sem_gb)
            fire_s(jb, 1, sem_sb)
            drain_s(ja, 0, sem_sa)
            drain_s(jb, 1, sem_sb)

        @pl.when(s < nextra)
        def _():
            pltpu.sync_copy(ei_hbm.at[0, pl.ds(16 * per + s, 1)], srcx_v)
            pltpu.sync_copy(ei_hbm.at[1, pl.ds(16 * per + s, 1)], dstx_v)
            pltpu.async_copy(
                yh.at[srcx_v.at[0]], rows_v.at[0], sem_ga).wait()
            pltpu.sync_copy(rows_v.at[0], acc.at[dstx_v.at[0]], add=True)

        plsc.subcore_barrier()
        pltpu.sync_copy(
            acc.at[pl.ds(s * rpt, rpt)],
            out_hbm.at[c, pl.ds(s * rpt, rpt)],
        )

    return edge


def _tc_mm_body(x_ref, w_ref, xw_ref):
    xw_ref[...] = jnp.dot(
        x_ref[...], w_ref[...], preferred_element_type=jnp.float32)


def _dinv_col(dp_ref, blk):
    dvec = dp_ref[0] + dp_ref[1] + 1.0
    return jnp.transpose(lax.rsqrt(dvec)[None, :])


def _tc_scale_body(xw_ref, dp_ref, y_ref):
    dinv = _dinv_col(dp_ref, xw_ref.shape[0])
    y = xw_ref[...] * dinv
    hd = y.shape[-1] // 2
    y_ref[0] = y[:, :hd]
    y_ref[1] = y[:, hd:]


def _tc_head_body(acc_ref, x_ref, dp_ref, bg_ref, w1_ref, b1_ref,
                  w2_ref, b2_ref, w3t_ref, b3_ref, o_ref):
    dinv = _dinv_col(dp_ref, x_ref.shape[0])
    acc = jnp.concatenate([acc_ref[0], acc_ref[1]], axis=1)
    tot = acc * dinv + bg_ref[...]
    h = jnp.maximum(tot, 0.0) + x_ref[...]
    h1 = jnp.dot(h, w1_ref[...], preferred_element_type=jnp.float32) + b1_ref[...]
    h1 = jnp.where(h1 >= 0.0, h1, 0.01 * h1)
    h2 = jnp.dot(h1, w2_ref[...], preferred_element_type=jnp.float32) + b2_ref[...]
    h2 = jnp.where(h2 >= 0.0, h2, 0.01 * h2)
    o = jnp.sum(h2 * w3t_ref[...], axis=1, keepdims=True) + b3_ref[...]
    o_ref[...] = o.reshape(o_ref.shape)


def kernel(x, edge_index, W_gcn, b_gcn, W1, b1, W2, b2, W3, b3):
    n, d = x.shape
    e = edge_index.shape[1]
    hd = d // 2
    nck = e // _CH
    rpt = 640
    n_acc = 16 * rpt
    act = 8

    ei3 = edge_index.astype(jnp.int32).reshape(2, nck, _CH)

    blk = 2048
    grid = n_acc // blk

    deg_p = _deg_kernel(n_acc, nck, rpt)(ei3)
    xw = pl.pallas_call(
        _tc_mm_body,
        grid=(grid,),
        in_specs=[
            pl.BlockSpec((blk, d), lambda i: (i, 0)),
            pl.BlockSpec((d, d), lambda i: (0, 0)),
        ],
        out_specs=pl.BlockSpec((blk, d), lambda i: (i, 0)),
        out_shape=jax.ShapeDtypeStruct((n, d), jnp.float32),
    )(x, W_gcn)

    y2 = pl.pallas_call(
        _tc_scale_body,
        grid=(grid,),
        in_specs=[
            pl.BlockSpec((blk, d), lambda i: (i, 0)),
            pl.BlockSpec((2, blk), lambda i: (0, i)),
        ],
        out_specs=pl.BlockSpec((2, blk, hd), lambda i: (0, i, 0)),
        out_shape=jax.ShapeDtypeStruct((2, n_acc, hd), jnp.float32),
    )(xw, deg_p)

    acc = _edge_kernel(n_acc, hd, nck, rpt)(y2, ei3)

    out = pl.pallas_call(
        _tc_head_body,
        grid=(grid,),
        in_specs=[
            pl.BlockSpec((2, blk, hd), lambda i: (0, i, 0)),
            pl.BlockSpec((blk, d), lambda i: (i, 0)),
            pl.BlockSpec((2, blk), lambda i: (0, i)),
            pl.BlockSpec((1, d), lambda i: (0, 0)),
            pl.BlockSpec((d, 32), lambda i: (0, 0)),
            pl.BlockSpec((1, 32), lambda i: (0, 0)),
            pl.BlockSpec((32, 32), lambda i: (0, 0)),
            pl.BlockSpec((1, 32), lambda i: (0, 0)),
            pl.BlockSpec((1, 32), lambda i: (0, 0)),
            pl.BlockSpec((1, 1), lambda i: (0, 0)),
        ],
        out_specs=pl.BlockSpec((blk // act, act), lambda i: (i, 0)),
        out_shape=jax.ShapeDtypeStruct((n // act, act), jnp.float32),
    )(acc, x, deg_p, b_gcn.reshape(1, d), W1, b1.reshape(1, 32),
      W2, b2.reshape(1, 32), W3.reshape(1, 32), b3.reshape(1, 1))

    return out.reshape(n // act, act, 1)
